# baseline (speedup 1.0000x reference)
import jax
import jax.numpy as jnp
from jax import lax
from jax.experimental import pallas as pl
from jax.experimental.pallas import tpu as pltpu

C = 64
D = 8
K = 15
NCODE = 16
OUT = 512
BLK = 512
CK16 = C * NCODE


def _body(x_ref, gall_ref, spad_ref, tpad_ref, hbd_ref, lut_ref, o_ref):
    xb16 = x_ref[...].astype(jnp.bfloat16).astype(jnp.float32)
    xall = jnp.dot(xb16, gall_ref[...],
                   preferred_element_type=jnp.float32)
    p1 = None
    for d in range(D):
        xd = xall[:, d * CK16:(d + 1) * CK16]
        sh = spad_ref[d:d + 1, :]
        term = xd * sh
        p1 = term if p1 is None else p1 + term
    p1 = p1 - tpad_ref[...]
    signs = jnp.where(p1 > 0, jnp.float32(1.0), jnp.float32(-1.0))
    p2 = jnp.dot(signs, hbd_ref[...], preferred_element_type=jnp.float32)
    p2_3 = p2.reshape(BLK, C, NCODE)
    codes = jnp.argmax(p2_3, axis=-1)
    iota = lax.broadcasted_iota(jnp.int32, (BLK, C, NCODE), 2)
    oh = (iota == codes[:, :, None]).astype(jnp.float32).reshape(BLK, CK16)
    o_ref[...] = jnp.dot(oh, lut_ref[...], preferred_element_type=jnp.float32)


def kernel(x, S, H, T, LUT):
    B = x.shape[0]
    cd = jnp.arange(C * D)
    c_of = cd // D
    d_of = cd % D
    kk = jnp.arange(K)
    cols = d_of[:, None] * CK16 + c_of[:, None] * NCODE + kk[None, :]
    gall = jnp.zeros((C * D, D * CK16), jnp.float32)
    gall = gall.at[jnp.repeat(cd, K), cols.reshape(-1)].set(1.0)
    s_t = jnp.pad(jnp.transpose(S, (1, 0, 2)), ((0, 0), (0, 0), (0, 1)))
    spad = s_t.reshape(D, CK16).astype(jnp.bfloat16).astype(jnp.float32)
    tpad = jnp.pad(T, ((0, 0), (0, 1))).reshape(1, CK16)
    h_pad = jnp.pad(H.astype(jnp.float32), ((0, 1), (0, 0)))
    hbd = jnp.kron(jnp.eye(C, dtype=jnp.float32), h_pad)
    lut2 = LUT.reshape(CK16, OUT)

    grid = (B // BLK,)
    return pl.pallas_call(
        _body,
        grid=grid,
        in_specs=[
            pl.BlockSpec((BLK, C * D), lambda i: (i, 0)),
            pl.BlockSpec((C * D, D * CK16), lambda i: (0, 0)),
            pl.BlockSpec((D, CK16), lambda i: (0, 0)),
            pl.BlockSpec((1, CK16), lambda i: (0, 0)),
            pl.BlockSpec((CK16, CK16), lambda i: (0, 0)),
            pl.BlockSpec((CK16, OUT), lambda i: (0, 0)),
        ],
        out_specs=pl.BlockSpec((BLK, OUT), lambda i: (i, 0)),
        out_shape=jax.ShapeDtypeStruct((B, OUT), jnp.float32),
        compiler_params=pltpu.CompilerParams(
            dimension_semantics=("arbitrary",),
        ),
    )(x, gall, spad, tpad, hbd, lut2)


# speedup vs baseline: 1.2818x; 1.2818x over previous
import jax
import jax.numpy as jnp
from jax import lax
from jax.experimental import pallas as pl
from jax.experimental.pallas import tpu as pltpu

C = 64
D = 8
K = 15
NCODE = 16
OUT = 512
BLK = 512
CK16 = C * NCODE


def _body(x_ref, sbd_ref, tpad_ref, hbd_ref, lut_ref, o_ref):
    p1 = jnp.dot(x_ref[...], sbd_ref[...],
                 preferred_element_type=jnp.float32)
    p1 = p1 - tpad_ref[...]
    signs = jnp.where(p1 > 0, jnp.float32(1.0), jnp.float32(-1.0))
    p2 = jnp.dot(signs, hbd_ref[...],
                 preferred_element_type=jnp.float32)
    p2_3 = p2.reshape(BLK, C, NCODE)
    codes = jnp.argmax(p2_3, axis=-1)
    iota = lax.broadcasted_iota(jnp.int32, (BLK, C, NCODE), 2)
    oh = (iota == codes[:, :, None]).astype(jnp.float32).reshape(BLK, CK16)
    o_ref[...] = jnp.dot(oh, lut_ref[...],
                         preferred_element_type=jnp.float32)


def kernel(x, S, H, T, LUT):
    B = x.shape[0]
    eye_c = jnp.eye(C, dtype=jnp.float32)
    s_pad = jnp.pad(S, ((0, 0), (0, 0), (0, 1)))
    sbd = (eye_c[:, None, :, None] * s_pad[:, :, None, :]).reshape(C * D, CK16)
    tpad = jnp.pad(T, ((0, 0), (0, 1))).reshape(1, CK16)
    h_pad = jnp.pad(H.astype(jnp.float32), ((0, 1), (0, 0)))
    hbd = jnp.kron(eye_c, h_pad)
    lut2 = LUT.reshape(CK16, OUT)

    grid = (B // BLK,)
    return pl.pallas_call(
        _body,
        grid=grid,
        in_specs=[
            pl.BlockSpec((BLK, C * D), lambda i: (i, 0)),
            pl.BlockSpec((C * D, CK16), lambda i: (0, 0)),
            pl.BlockSpec((1, CK16), lambda i: (0, 0)),
            pl.BlockSpec((CK16, CK16), lambda i: (0, 0)),
            pl.BlockSpec((CK16, OUT), lambda i: (0, 0)),
        ],
        out_specs=pl.BlockSpec((BLK, OUT), lambda i: (i, 0)),
        out_shape=jax.ShapeDtypeStruct((B, OUT), jnp.float32),
        compiler_params=pltpu.CompilerParams(
            dimension_semantics=("arbitrary",),
        ),
    )(x, sbd, tpad, hbd, lut2)
